# baseline (speedup 1.0000x reference)
import functools

import jax
import jax.numpy as jnp
from jax import lax
from jax.experimental import pallas as pl
from jax.experimental.pallas import tpu as pltpu
from jax.experimental.pallas import tpu_sc as plsc

_NW = 32
_C = 512


def _dequant_body(ids_hbm, qw_hbm, sc_hbm, out_hbm,
                  idx_v, rows_v, scl_v, out_v, sem, *, per_w, n_chunks):
    wid = lax.axis_index("s") * 2 + lax.axis_index("c")
    base = wid * per_w
    lane = lax.iota(jnp.int32, 16)
    col4 = [lane * 4 + j for j in range(4)]

    def chunk_body(ci, _):
        off = base + ci * _C
        pltpu.sync_copy(ids_hbm.at[pl.ds(off, _C)], idx_v)
        cp_rows = pltpu.async_copy(qw_hbm.at[idx_v], rows_v, sem)
        cp_scl = pltpu.async_copy(sc_hbm.at[idx_v], scl_v, sem)
        cp_rows.wait()
        cp_scl.wait()

        def group_body(g, _):
            svec = scl_v[pl.ds(g * 16, 16)]
            for r in range(16):
                i = g * 16 + r
                words = rows_v[i, :]
                s = jnp.full((16,), svec[r], jnp.float32)
                rbase = jnp.full((16,), i * 64, jnp.int32)
                for j in range(4):
                    b = (words << (24 - 8 * j)) >> 24
                    plsc.store_scatter(out_v, [rbase + col4[j]],
                                       b.astype(jnp.float32) * s)
            return 0

        lax.fori_loop(0, _C // 16, group_body, 0)
        pltpu.sync_copy(out_v, out_hbm.at[pl.ds(off * 64, _C * 64)])
        return 0

    lax.fori_loop(0, n_chunks, chunk_body, 0)


def kernel(input_ids, q_weight, scale):
    B, L = input_ids.shape
    V, D = q_weight.shape
    N = B * L
    per_w = N // _NW
    n_chunks = per_w // _C
    assert per_w * _NW == N and n_chunks * _C == per_w

    ids = input_ids.reshape(N).astype(jnp.int32)
    scale1 = scale.reshape(V)
    qw32 = lax.bitcast_convert_type(q_weight.reshape(V, D // 4, 4), jnp.int32)

    mesh = plsc.VectorSubcoreMesh(core_axis_name="c", subcore_axis_name="s")
    run = pl.kernel(
        functools.partial(_dequant_body, per_w=per_w, n_chunks=n_chunks),
        out_type=jax.ShapeDtypeStruct((N * D,), jnp.float32),
        mesh=mesh,
        scratch_types=[
            pltpu.VMEM((_C,), jnp.int32),
            pltpu.VMEM((_C, D // 4), jnp.int32),
            pltpu.VMEM((_C,), jnp.float32),
            pltpu.VMEM((_C * D,), jnp.float32),
            pltpu.SemaphoreType.DMA,
        ],
        compiler_params=pltpu.CompilerParams(
            needs_layout_passes=False, use_tc_tiling_on_sc=False),
    )
    out = run(ids, qw32, scale1)
    return out.reshape(B, L, D)
